# baseline (speedup 1.0000x reference)
import functools

import jax
import jax.numpy as jnp
from jax import lax
from jax.experimental import pallas as pl
from jax.experimental.pallas import tpu as pltpu
from jax.experimental.pallas import tpu_sc as plsc

N = 10000
E = 320000
H = 128
BN_EPS = 1e-5

NC = 2
NS = 16
NW = NC * NS
EDGES_PER_W = E // NW
CHUNK = 100
CHUNKS = EDGES_PER_W // CHUNK
HALF = CHUNKS // 2
N_PAD = 10240
ROWS_PER_TILE = N_PAD // NS

_mesh = plsc.VectorSubcoreMesh(core_axis_name="c", subcore_axis_name="s")


@functools.partial(
    pl.kernel,
    out_type=jax.ShapeDtypeStruct((NC, N_PAD, H), jnp.float32),
    mesh=_mesh,
    scratch_types=[
        pltpu.VMEM((2, HALF, CHUNK), jnp.int32),
        pltpu.VMEM((CHUNK, H), jnp.float32),
        pltpu.VMEM((CHUNK, H), jnp.float32),
        pltpu.VMEM_SHARED((N_PAD, H), jnp.float32),
        pltpu.SemaphoreType.DMA,
        pltpu.SemaphoreType.DMA,
        pltpu.SemaphoreType.DMA,
        pltpu.SemaphoreType.DMA,
    ],
)
def _sc_segment_sum(h_hbm, idx_hbm, zeros_hbm, out_hbm,
                    idx_v, rows0_v, rows1_v, acc_sh,
                    sem_g0, sem_g1, sem_s0, sem_s1):
    cid = lax.axis_index("c")
    sid = lax.axis_index("s")
    wid = sid * NC + cid

    pltpu.sync_copy(zeros_hbm, acc_sh.at[pl.ds(sid * ROWS_PER_TILE, ROWS_PER_TILE)])
    plsc.subcore_barrier()

    for p in range(2):
        pltpu.sync_copy(idx_hbm.at[wid, p], idx_v)
        pltpu.async_copy(h_hbm.at[idx_v.at[0, 0]], rows0_v, sem_g0)
        pltpu.async_copy(h_hbm.at[idx_v.at[0, 1]], rows1_v, sem_g1)

        def body(i, carry):
            j = i * 2
            pltpu.make_async_copy(h_hbm.at[idx_v.at[0, 0]], rows0_v, sem_g0).wait()
            pltpu.async_copy(rows0_v, acc_sh.at[idx_v.at[1, j]], sem_s0, add=True)
            pltpu.make_async_copy(h_hbm.at[idx_v.at[0, 0]], rows1_v, sem_g1).wait()
            pltpu.async_copy(rows1_v, acc_sh.at[idx_v.at[1, j + 1]], sem_s1, add=True)

            pltpu.make_async_copy(rows0_v, acc_sh.at[idx_v.at[1, 0]], sem_s0).wait()

            @pl.when(j + 2 < HALF)
            def _():
                pltpu.async_copy(h_hbm.at[idx_v.at[0, j + 2]], rows0_v, sem_g0)

            pltpu.make_async_copy(rows1_v, acc_sh.at[idx_v.at[1, 0]], sem_s1).wait()

            @pl.when(j + 3 < HALF)
            def _():
                pltpu.async_copy(h_hbm.at[idx_v.at[0, j + 3]], rows1_v, sem_g1)

            return carry

        lax.fori_loop(0, HALF // 2, body, 0)

    plsc.subcore_barrier()
    pltpu.sync_copy(acc_sh.at[pl.ds(sid * ROWS_PER_TILE, ROWS_PER_TILE)],
                    out_hbm.at[cid, pl.ds(sid * ROWS_PER_TILE, ROWS_PER_TILE)])


ROW_BLK = 1000


def _mlp_body(h_ref, agg_ref, w1_ref, b1_ref, w2_ref, b2_ref, o_ref):
    z = h_ref[...] + agg_ref[0] + agg_ref[1]
    t = jnp.dot(z, w1_ref[...], preferred_element_type=jnp.float32) + b1_ref[...]
    t = jnp.maximum(t, 0.0)
    t = jnp.dot(t, w2_ref[...], preferred_element_type=jnp.float32) + b2_ref[...]
    o_ref[...] = jnp.maximum(t, 0.0)


def _tc_mlp(h, agg, W1, b1, W2, b2):
    return pl.pallas_call(
        _mlp_body,
        grid=(N // ROW_BLK,),
        in_specs=[
            pl.BlockSpec((ROW_BLK, H), lambda i: (i, 0)),
            pl.BlockSpec((NC, ROW_BLK, H), lambda i: (0, i, 0)),
            pl.BlockSpec((H, H), lambda i: (0, 0)),
            pl.BlockSpec((1, H), lambda i: (0, 0)),
            pl.BlockSpec((H, H), lambda i: (0, 0)),
            pl.BlockSpec((1, H), lambda i: (0, 0)),
        ],
        out_specs=pl.BlockSpec((ROW_BLK, H), lambda i: (i, 0)),
        out_shape=jax.ShapeDtypeStruct((N, H), jnp.float32),
    )(h, agg, W1, b1.reshape(1, H), W2, b2.reshape(1, H))


def _mlp3_head_body(h_ref, agg_ref, w1_ref, b1_ref, w2_ref, b2_ref,
                    h1_ref, h2_ref, wjk_ref, bjk_ref, wc1_ref, bc1_ref,
                    gam_ref, bet_ref, mu_ref, var_ref, wc2_ref, bc2_ref, o_ref):
    z = h_ref[...] + agg_ref[0] + agg_ref[1]
    t = jnp.dot(z, w1_ref[...], preferred_element_type=jnp.float32) + b1_ref[...]
    t = jnp.maximum(t, 0.0)
    t = jnp.dot(t, w2_ref[...], preferred_element_type=jnp.float32) + b2_ref[...]
    h3 = jnp.maximum(t, 0.0)
    t = jnp.dot(h1_ref[...], wjk_ref[0:H, :], preferred_element_type=jnp.float32)
    t += jnp.dot(h2_ref[...], wjk_ref[H:2 * H, :], preferred_element_type=jnp.float32)
    t += jnp.dot(h3, wjk_ref[2 * H:3 * H, :], preferred_element_type=jnp.float32)
    t += bjk_ref[...]
    u = jnp.dot(t, wc1_ref[...], preferred_element_type=jnp.float32) + bc1_ref[...]
    scale = gam_ref[...] * lax.rsqrt(var_ref[...] + BN_EPS)
    u = (u - mu_ref[...]) * scale + bet_ref[...]
    u = jnp.maximum(u, 0.0)
    o_ref[...] = jnp.dot(u, wc2_ref[...], preferred_element_type=jnp.float32) + bc2_ref[...]


def _tc_mlp3_head(h, agg, W1, b1, W2, b2, h1, h2, W_jk, b_jk, Wc1, bc1,
                  gam, bet, mu, var, Wc2, bc2):
    row = lambda v: v.reshape(1, H)
    full = lambda shape: pl.BlockSpec(shape, lambda i: (0,) * len(shape))
    blk = pl.BlockSpec((ROW_BLK, H), lambda i: (i, 0))
    return pl.pallas_call(
        _mlp3_head_body,
        grid=(N // ROW_BLK,),
        in_specs=[blk,
                  pl.BlockSpec((NC, ROW_BLK, H), lambda i: (0, i, 0)),
                  full((H, H)), full((1, H)), full((H, H)), full((1, H)),
                  blk, blk,
                  full((3 * H, H)), full((1, H)),
                  full((H, H)), full((1, H)),
                  full((1, H)), full((1, H)), full((1, H)), full((1, H)),
                  full((H, H)), full((1, H))],
        out_specs=blk,
        out_shape=jax.ShapeDtypeStruct((N, H), jnp.float32),
    )(h, agg, W1, row(b1), W2, row(b2), h1, h2, W_jk, row(b_jk), Wc1, row(bc1),
      row(gam), row(bet), row(mu), row(var), Wc2, row(bc2))


def kernel(x, edge_index, W1_0, b1_0, W2_0, b2_0, W1_1, b1_1, W2_1, b2_1,
           W1_2, b1_2, W2_2, b2_2, W_jk, b_jk, Wc1, bc1,
           bn_gamma, bn_beta, bn_mean, bn_var, Wc2, bc2):
    idx = edge_index.reshape(2, NW, 2, HALF, CHUNK).transpose(1, 2, 0, 3, 4)
    zeros = jnp.zeros((ROWS_PER_TILE, H), jnp.float32)

    h = x
    xs = []
    for (W1, b1, W2, b2) in ((W1_0, b1_0, W2_0, b2_0),
                             (W1_1, b1_1, W2_1, b2_1)):
        agg = _sc_segment_sum(h, idx, zeros)
        h = _tc_mlp(h, agg, W1, b1, W2, b2)
        xs.append(h)

    agg = _sc_segment_sum(h, idx, zeros)
    return _tc_mlp3_head(h, agg, W1_2, b1_2, W2_2, b2_2, xs[0], xs[1],
                         W_jk, b_jk, Wc1, bc1,
                         bn_gamma, bn_beta, bn_mean, bn_var, Wc2, bc2)


# speedup vs baseline: 1.2698x; 1.2698x over previous
import functools

import jax
import jax.numpy as jnp
from jax import lax
from jax.experimental import pallas as pl
from jax.experimental.pallas import tpu as pltpu
from jax.experimental.pallas import tpu_sc as plsc

N = 10000
E = 320000
H = 128
BN_EPS = 1e-5

NC = 2
NS = 16
NW = NC * NS
EDGES_PER_W = E // NW
CHUNK = 100
CHUNKS = EDGES_PER_W // CHUNK
HALF = CHUNKS // 2
N_PAD = 10240
ROWS_PER_TILE = N_PAD // NS

_mesh = plsc.VectorSubcoreMesh(core_axis_name="c", subcore_axis_name="s")


@functools.partial(
    pl.kernel,
    out_type=jax.ShapeDtypeStruct((NC, N_PAD, H), jnp.float32),
    mesh=_mesh,
    scratch_types=[
        pltpu.VMEM((2, HALF, CHUNK), jnp.int32),
        pltpu.VMEM((CHUNK, H), jnp.float32),
        pltpu.VMEM((CHUNK, H), jnp.float32),
        pltpu.VMEM_SHARED((N_PAD, H), jnp.float32),
        pltpu.SemaphoreType.DMA,
        pltpu.SemaphoreType.DMA,
        pltpu.SemaphoreType.DMA,
        pltpu.SemaphoreType.DMA,
    ],
)
def _sc_segment_sum(h_hbm, idx_hbm, zeros_hbm, out_hbm,
                    idx_v, rows0_v, rows1_v, acc_sh,
                    sem_g0, sem_g1, sem_s0, sem_s1):
    cid = lax.axis_index("c")
    sid = lax.axis_index("s")
    wid = sid * NC + cid

    pltpu.sync_copy(zeros_hbm, acc_sh.at[pl.ds(sid * ROWS_PER_TILE, ROWS_PER_TILE)])
    plsc.subcore_barrier()

    for p in range(2):
        pltpu.sync_copy(idx_hbm.at[wid, p], idx_v)
        pltpu.async_copy(h_hbm.at[idx_v.at[0, 0]], rows0_v, sem_g0)
        pltpu.async_copy(h_hbm.at[idx_v.at[0, 1]], rows1_v, sem_g1)

        def body(i, carry):
            j = i * 2
            pltpu.make_async_copy(h_hbm.at[idx_v.at[0, 0]], rows0_v, sem_g0).wait()
            pltpu.sync_copy(rows0_v, acc_sh.at[idx_v.at[1, j]], add=True)

            @pl.when(j + 2 < HALF)
            def _():
                pltpu.async_copy(h_hbm.at[idx_v.at[0, j + 2]], rows0_v, sem_g0)

            pltpu.make_async_copy(h_hbm.at[idx_v.at[0, 0]], rows1_v, sem_g1).wait()
            pltpu.sync_copy(rows1_v, acc_sh.at[idx_v.at[1, j + 1]], add=True)

            @pl.when(j + 3 < HALF)
            def _():
                pltpu.async_copy(h_hbm.at[idx_v.at[0, j + 3]], rows1_v, sem_g1)

            return carry

        lax.fori_loop(0, HALF // 2, body, 0)

    plsc.subcore_barrier()
    pltpu.sync_copy(acc_sh.at[pl.ds(sid * ROWS_PER_TILE, ROWS_PER_TILE)],
                    out_hbm.at[cid, pl.ds(sid * ROWS_PER_TILE, ROWS_PER_TILE)])


ROW_BLK = 1000


def _mlp_body(h_ref, agg_ref, w1_ref, b1_ref, w2_ref, b2_ref, o_ref):
    z = h_ref[...] + agg_ref[0] + agg_ref[1]
    t = jnp.dot(z, w1_ref[...], preferred_element_type=jnp.float32) + b1_ref[...]
    t = jnp.maximum(t, 0.0)
    t = jnp.dot(t, w2_ref[...], preferred_element_type=jnp.float32) + b2_ref[...]
    o_ref[...] = jnp.maximum(t, 0.0)


def _tc_mlp(h, agg, W1, b1, W2, b2):
    return pl.pallas_call(
        _mlp_body,
        grid=(N // ROW_BLK,),
        in_specs=[
            pl.BlockSpec((ROW_BLK, H), lambda i: (i, 0)),
            pl.BlockSpec((NC, ROW_BLK, H), lambda i: (0, i, 0)),
            pl.BlockSpec((H, H), lambda i: (0, 0)),
            pl.BlockSpec((1, H), lambda i: (0, 0)),
            pl.BlockSpec((H, H), lambda i: (0, 0)),
            pl.BlockSpec((1, H), lambda i: (0, 0)),
        ],
        out_specs=pl.BlockSpec((ROW_BLK, H), lambda i: (i, 0)),
        out_shape=jax.ShapeDtypeStruct((N, H), jnp.float32),
    )(h, agg, W1, b1.reshape(1, H), W2, b2.reshape(1, H))


def _mlp3_head_body(h_ref, agg_ref, w1_ref, b1_ref, w2_ref, b2_ref,
                    h1_ref, h2_ref, wjk_ref, bjk_ref, wc1_ref, bc1_ref,
                    gam_ref, bet_ref, mu_ref, var_ref, wc2_ref, bc2_ref, o_ref):
    z = h_ref[...] + agg_ref[0] + agg_ref[1]
    t = jnp.dot(z, w1_ref[...], preferred_element_type=jnp.float32) + b1_ref[...]
    t = jnp.maximum(t, 0.0)
    t = jnp.dot(t, w2_ref[...], preferred_element_type=jnp.float32) + b2_ref[...]
    h3 = jnp.maximum(t, 0.0)
    t = jnp.dot(h1_ref[...], wjk_ref[0:H, :], preferred_element_type=jnp.float32)
    t += jnp.dot(h2_ref[...], wjk_ref[H:2 * H, :], preferred_element_type=jnp.float32)
    t += jnp.dot(h3, wjk_ref[2 * H:3 * H, :], preferred_element_type=jnp.float32)
    t += bjk_ref[...]
    u = jnp.dot(t, wc1_ref[...], preferred_element_type=jnp.float32) + bc1_ref[...]
    scale = gam_ref[...] * lax.rsqrt(var_ref[...] + BN_EPS)
    u = (u - mu_ref[...]) * scale + bet_ref[...]
    u = jnp.maximum(u, 0.0)
    o_ref[...] = jnp.dot(u, wc2_ref[...], preferred_element_type=jnp.float32) + bc2_ref[...]


def _tc_mlp3_head(h, agg, W1, b1, W2, b2, h1, h2, W_jk, b_jk, Wc1, bc1,
                  gam, bet, mu, var, Wc2, bc2):
    row = lambda v: v.reshape(1, H)
    full = lambda shape: pl.BlockSpec(shape, lambda i: (0,) * len(shape))
    blk = pl.BlockSpec((ROW_BLK, H), lambda i: (i, 0))
    return pl.pallas_call(
        _mlp3_head_body,
        grid=(N // ROW_BLK,),
        in_specs=[blk,
                  pl.BlockSpec((NC, ROW_BLK, H), lambda i: (0, i, 0)),
                  full((H, H)), full((1, H)), full((H, H)), full((1, H)),
                  blk, blk,
                  full((3 * H, H)), full((1, H)),
                  full((H, H)), full((1, H)),
                  full((1, H)), full((1, H)), full((1, H)), full((1, H)),
                  full((H, H)), full((1, H))],
        out_specs=blk,
        out_shape=jax.ShapeDtypeStruct((N, H), jnp.float32),
    )(h, agg, W1, row(b1), W2, row(b2), h1, h2, W_jk, row(b_jk), Wc1, row(bc1),
      row(gam), row(bet), row(mu), row(var), Wc2, row(bc2))


def kernel(x, edge_index, W1_0, b1_0, W2_0, b2_0, W1_1, b1_1, W2_1, b2_1,
           W1_2, b1_2, W2_2, b2_2, W_jk, b_jk, Wc1, bc1,
           bn_gamma, bn_beta, bn_mean, bn_var, Wc2, bc2):
    idx = edge_index.reshape(2, NW, 2, HALF, CHUNK).transpose(1, 2, 0, 3, 4)
    zeros = jnp.zeros((ROWS_PER_TILE, H), jnp.float32)

    h = x
    xs = []
    for (W1, b1, W2, b2) in ((W1_0, b1_0, W2_0, b2_0),
                             (W1_1, b1_1, W2_1, b2_1)):
        agg = _sc_segment_sum(h, idx, zeros)
        h = _tc_mlp(h, agg, W1, b1, W2, b2)
        xs.append(h)

    agg = _sc_segment_sum(h, idx, zeros)
    return _tc_mlp3_head(h, agg, W1_2, b1_2, W2_2, b2_2, xs[0], xs[1],
                         W_jk, b_jk, Wc1, bc1,
                         bn_gamma, bn_beta, bn_mean, bn_var, Wc2, bc2)


# speedup vs baseline: 1.3054x; 1.0281x over previous
import functools

import jax
import jax.numpy as jnp
from jax import lax
from jax.experimental import pallas as pl
from jax.experimental.pallas import tpu as pltpu
from jax.experimental.pallas import tpu_sc as plsc

N = 10000
E = 320000
H = 128
BN_EPS = 1e-5

NC = 2
NS = 16
NW = NC * NS
EDGES_PER_W = E // NW
CHUNK = 125
CHUNKS = EDGES_PER_W // CHUNK
HALF = CHUNKS // 2
N_PAD = 10240
ROWS_PER_TILE = N_PAD // NS

_mesh = plsc.VectorSubcoreMesh(core_axis_name="c", subcore_axis_name="s")


@functools.partial(
    pl.kernel,
    out_type=jax.ShapeDtypeStruct((NC, N_PAD, H), jnp.float32),
    mesh=_mesh,
    scratch_types=[
        pltpu.VMEM((2, HALF, CHUNK), jnp.int32),
        pltpu.VMEM((CHUNK, H), jnp.float32),
        pltpu.VMEM((CHUNK, H), jnp.float32),
        pltpu.VMEM_SHARED((N_PAD, H), jnp.float32),
        pltpu.SemaphoreType.DMA,
        pltpu.SemaphoreType.DMA,
    ],
)
def _sc_segment_sum(h_hbm, idx_hbm, zeros_hbm, out_hbm,
                    idx_v, rows0_v, rows1_v, acc_sh, sem_g0, sem_g1):
    cid = lax.axis_index("c")
    sid = lax.axis_index("s")
    wid = sid * NC + cid

    pltpu.sync_copy(zeros_hbm, acc_sh.at[pl.ds(sid * ROWS_PER_TILE, ROWS_PER_TILE)])
    plsc.subcore_barrier()

    for p in range(2):
        pltpu.sync_copy(idx_hbm.at[wid, p], idx_v)
        pltpu.async_copy(h_hbm.at[idx_v.at[0, 0]], rows0_v, sem_g0)
        pltpu.async_copy(h_hbm.at[idx_v.at[0, 1]], rows1_v, sem_g1)

        def body(i, carry):
            j = i * 2
            pltpu.make_async_copy(h_hbm.at[idx_v.at[0, 0]], rows0_v, sem_g0).wait()
            pltpu.sync_copy(rows0_v, acc_sh.at[idx_v.at[1, j]], add=True)

            @pl.when(j + 2 < HALF)
            def _():
                pltpu.async_copy(h_hbm.at[idx_v.at[0, j + 2]], rows0_v, sem_g0)

            pltpu.make_async_copy(h_hbm.at[idx_v.at[0, 0]], rows1_v, sem_g1).wait()
            pltpu.sync_copy(rows1_v, acc_sh.at[idx_v.at[1, j + 1]], add=True)

            @pl.when(j + 3 < HALF)
            def _():
                pltpu.async_copy(h_hbm.at[idx_v.at[0, j + 3]], rows1_v, sem_g1)

            return carry

        lax.fori_loop(0, HALF // 2, body, 0)

    plsc.subcore_barrier()
    pltpu.sync_copy(acc_sh.at[pl.ds(sid * ROWS_PER_TILE, ROWS_PER_TILE)],
                    out_hbm.at[cid, pl.ds(sid * ROWS_PER_TILE, ROWS_PER_TILE)])


ROW_BLK = 1000


def _mlp_body(h_ref, agg_ref, w1_ref, b1_ref, w2_ref, b2_ref, o_ref):
    z = h_ref[...] + agg_ref[0] + agg_ref[1]
    t = jnp.dot(z, w1_ref[...], preferred_element_type=jnp.float32) + b1_ref[...]
    t = jnp.maximum(t, 0.0)
    t = jnp.dot(t, w2_ref[...], preferred_element_type=jnp.float32) + b2_ref[...]
    o_ref[...] = jnp.maximum(t, 0.0)


def _tc_mlp(h, agg, W1, b1, W2, b2):
    return pl.pallas_call(
        _mlp_body,
        grid=(N // ROW_BLK,),
        in_specs=[
            pl.BlockSpec((ROW_BLK, H), lambda i: (i, 0)),
            pl.BlockSpec((NC, ROW_BLK, H), lambda i: (0, i, 0)),
            pl.BlockSpec((H, H), lambda i: (0, 0)),
            pl.BlockSpec((1, H), lambda i: (0, 0)),
            pl.BlockSpec((H, H), lambda i: (0, 0)),
            pl.BlockSpec((1, H), lambda i: (0, 0)),
        ],
        out_specs=pl.BlockSpec((ROW_BLK, H), lambda i: (i, 0)),
        out_shape=jax.ShapeDtypeStruct((N, H), jnp.float32),
    )(h, agg, W1, b1.reshape(1, H), W2, b2.reshape(1, H))


def _mlp3_head_body(h_ref, agg_ref, w1_ref, b1_ref, w2_ref, b2_ref,
                    h1_ref, h2_ref, wjk_ref, bjk_ref, wc1_ref, bc1_ref,
                    gam_ref, bet_ref, mu_ref, var_ref, wc2_ref, bc2_ref, o_ref):
    z = h_ref[...] + agg_ref[0] + agg_ref[1]
    t = jnp.dot(z, w1_ref[...], preferred_element_type=jnp.float32) + b1_ref[...]
    t = jnp.maximum(t, 0.0)
    t = jnp.dot(t, w2_ref[...], preferred_element_type=jnp.float32) + b2_ref[...]
    h3 = jnp.maximum(t, 0.0)
    t = jnp.dot(h1_ref[...], wjk_ref[0:H, :], preferred_element_type=jnp.float32)
    t += jnp.dot(h2_ref[...], wjk_ref[H:2 * H, :], preferred_element_type=jnp.float32)
    t += jnp.dot(h3, wjk_ref[2 * H:3 * H, :], preferred_element_type=jnp.float32)
    t += bjk_ref[...]
    u = jnp.dot(t, wc1_ref[...], preferred_element_type=jnp.float32) + bc1_ref[...]
    scale = gam_ref[...] * lax.rsqrt(var_ref[...] + BN_EPS)
    u = (u - mu_ref[...]) * scale + bet_ref[...]
    u = jnp.maximum(u, 0.0)
    o_ref[...] = jnp.dot(u, wc2_ref[...], preferred_element_type=jnp.float32) + bc2_ref[...]


def _tc_mlp3_head(h, agg, W1, b1, W2, b2, h1, h2, W_jk, b_jk, Wc1, bc1,
                  gam, bet, mu, var, Wc2, bc2):
    row = lambda v: v.reshape(1, H)
    full = lambda shape: pl.BlockSpec(shape, lambda i: (0,) * len(shape))
    blk = pl.BlockSpec((ROW_BLK, H), lambda i: (i, 0))
    return pl.pallas_call(
        _mlp3_head_body,
        grid=(N // ROW_BLK,),
        in_specs=[blk,
                  pl.BlockSpec((NC, ROW_BLK, H), lambda i: (0, i, 0)),
                  full((H, H)), full((1, H)), full((H, H)), full((1, H)),
                  blk, blk,
                  full((3 * H, H)), full((1, H)),
                  full((H, H)), full((1, H)),
                  full((1, H)), full((1, H)), full((1, H)), full((1, H)),
                  full((H, H)), full((1, H))],
        out_specs=blk,
        out_shape=jax.ShapeDtypeStruct((N, H), jnp.float32),
    )(h, agg, W1, row(b1), W2, row(b2), h1, h2, W_jk, row(b_jk), Wc1, row(bc1),
      row(gam), row(bet), row(mu), row(var), Wc2, row(bc2))


def kernel(x, edge_index, W1_0, b1_0, W2_0, b2_0, W1_1, b1_1, W2_1, b2_1,
           W1_2, b1_2, W2_2, b2_2, W_jk, b_jk, Wc1, bc1,
           bn_gamma, bn_beta, bn_mean, bn_var, Wc2, bc2):
    idx = edge_index.reshape(2, NW, 2, HALF, CHUNK).transpose(1, 2, 0, 3, 4)
    zeros = jnp.zeros((ROWS_PER_TILE, H), jnp.float32)

    h = x
    xs = []
    for (W1, b1, W2, b2) in ((W1_0, b1_0, W2_0, b2_0),
                             (W1_1, b1_1, W2_1, b2_1)):
        agg = _sc_segment_sum(h, idx, zeros)
        h = _tc_mlp(h, agg, W1, b1, W2, b2)
        xs.append(h)

    agg = _sc_segment_sum(h, idx, zeros)
    return _tc_mlp3_head(h, agg, W1_2, b1_2, W2_2, b2_2, xs[0], xs[1],
                         W_jk, b_jk, Wc1, bc1,
                         bn_gamma, bn_beta, bn_mean, bn_var, Wc2, bc2)


# speedup vs baseline: 1.3258x; 1.0156x over previous
import functools

import jax
import jax.numpy as jnp
from jax import lax
from jax.experimental import pallas as pl
from jax.experimental.pallas import tpu as pltpu
from jax.experimental.pallas import tpu_sc as plsc

N = 10000
E = 320000
H = 128
BN_EPS = 1e-5

NC = 2
NS = 16
NW = NC * NS
EDGES_PER_W = E // NW
CHUNK = 125
CHUNKS = EDGES_PER_W // CHUNK
HALF = CHUNKS // 2
N_PAD = 10240
ROWS_PER_TILE = N_PAD // NS

_mesh = plsc.VectorSubcoreMesh(core_axis_name="c", subcore_axis_name="s")


@functools.partial(
    pl.kernel,
    out_type=jax.ShapeDtypeStruct((NC, N_PAD, H), jnp.float32),
    mesh=_mesh,
    scratch_types=[
        pltpu.VMEM((HALF, CHUNK), jnp.int32),
        pltpu.VMEM((HALF, CHUNK), jnp.int32),
        pltpu.VMEM((CHUNK, H), jnp.float32),
        pltpu.VMEM((CHUNK, H), jnp.float32),
        pltpu.VMEM_SHARED((N_PAD, H), jnp.float32),
        pltpu.SemaphoreType.DMA,
        pltpu.SemaphoreType.DMA,
    ],
)
def _sc_segment_sum(h_hbm, idx_hbm, zeros_hbm, out_hbm,
                    src_v, dst_v, rows0_v, rows1_v, acc_sh, sem_g0, sem_g1):
    cid = lax.axis_index("c")
    sid = lax.axis_index("s")
    wid = sid * NC + cid

    pltpu.sync_copy(zeros_hbm, acc_sh.at[pl.ds(sid * ROWS_PER_TILE, ROWS_PER_TILE)])
    plsc.subcore_barrier()

    for p in range(2):
        pltpu.sync_copy(idx_hbm.at[0, wid, p], src_v)
        pltpu.sync_copy(idx_hbm.at[1, wid, p], dst_v)
        pltpu.async_copy(h_hbm.at[src_v.at[0]], rows0_v, sem_g0)
        pltpu.async_copy(h_hbm.at[src_v.at[1]], rows1_v, sem_g1)

        def body(i, carry):
            j = i * 2
            pltpu.make_async_copy(h_hbm.at[src_v.at[0]], rows0_v, sem_g0).wait()
            pltpu.sync_copy(rows0_v, acc_sh.at[dst_v.at[j]], add=True)

            @pl.when(j + 2 < HALF)
            def _():
                pltpu.async_copy(h_hbm.at[src_v.at[j + 2]], rows0_v, sem_g0)

            pltpu.make_async_copy(h_hbm.at[src_v.at[0]], rows1_v, sem_g1).wait()
            pltpu.sync_copy(rows1_v, acc_sh.at[dst_v.at[j + 1]], add=True)

            @pl.when(j + 3 < HALF)
            def _():
                pltpu.async_copy(h_hbm.at[src_v.at[j + 3]], rows1_v, sem_g1)

            return carry

        lax.fori_loop(0, HALF // 2, body, 0)

    plsc.subcore_barrier()
    pltpu.sync_copy(acc_sh.at[pl.ds(sid * ROWS_PER_TILE, ROWS_PER_TILE)],
                    out_hbm.at[cid, pl.ds(sid * ROWS_PER_TILE, ROWS_PER_TILE)])


ROW_BLK = 2000


def _mlp_body(h_ref, agg_ref, w1_ref, b1_ref, w2_ref, b2_ref, o_ref):
    z = h_ref[...] + agg_ref[0] + agg_ref[1]
    t = jnp.dot(z, w1_ref[...], preferred_element_type=jnp.float32) + b1_ref[...]
    t = jnp.maximum(t, 0.0)
    t = jnp.dot(t, w2_ref[...], preferred_element_type=jnp.float32) + b2_ref[...]
    o_ref[...] = jnp.maximum(t, 0.0)


def _tc_mlp(h, agg, W1, b1, W2, b2):
    return pl.pallas_call(
        _mlp_body,
        grid=(N // ROW_BLK,),
        in_specs=[
            pl.BlockSpec((ROW_BLK, H), lambda i: (i, 0)),
            pl.BlockSpec((NC, ROW_BLK, H), lambda i: (0, i, 0)),
            pl.BlockSpec((H, H), lambda i: (0, 0)),
            pl.BlockSpec((1, H), lambda i: (0, 0)),
            pl.BlockSpec((H, H), lambda i: (0, 0)),
            pl.BlockSpec((1, H), lambda i: (0, 0)),
        ],
        out_specs=pl.BlockSpec((ROW_BLK, H), lambda i: (i, 0)),
        out_shape=jax.ShapeDtypeStruct((N, H), jnp.float32),
    )(h, agg, W1, b1.reshape(1, H), W2, b2.reshape(1, H))


def _mlp3_head_body(h_ref, agg_ref, w1_ref, b1_ref, w2_ref, b2_ref,
                    h1_ref, h2_ref, wjk_ref, bjk_ref, wc1_ref, bc1_ref,
                    gam_ref, bet_ref, mu_ref, var_ref, wc2_ref, bc2_ref, o_ref):
    z = h_ref[...] + agg_ref[0] + agg_ref[1]
    t = jnp.dot(z, w1_ref[...], preferred_element_type=jnp.float32) + b1_ref[...]
    t = jnp.maximum(t, 0.0)
    t = jnp.dot(t, w2_ref[...], preferred_element_type=jnp.float32) + b2_ref[...]
    h3 = jnp.maximum(t, 0.0)
    t = jnp.dot(h1_ref[...], wjk_ref[0:H, :], preferred_element_type=jnp.float32)
    t += jnp.dot(h2_ref[...], wjk_ref[H:2 * H, :], preferred_element_type=jnp.float32)
    t += jnp.dot(h3, wjk_ref[2 * H:3 * H, :], preferred_element_type=jnp.float32)
    t += bjk_ref[...]
    u = jnp.dot(t, wc1_ref[...], preferred_element_type=jnp.float32) + bc1_ref[...]
    scale = gam_ref[...] * lax.rsqrt(var_ref[...] + BN_EPS)
    u = (u - mu_ref[...]) * scale + bet_ref[...]
    u = jnp.maximum(u, 0.0)
    o_ref[...] = jnp.dot(u, wc2_ref[...], preferred_element_type=jnp.float32) + bc2_ref[...]


def _tc_mlp3_head(h, agg, W1, b1, W2, b2, h1, h2, W_jk, b_jk, Wc1, bc1,
                  gam, bet, mu, var, Wc2, bc2):
    row = lambda v: v.reshape(1, H)
    full = lambda shape: pl.BlockSpec(shape, lambda i: (0,) * len(shape))
    blk = pl.BlockSpec((ROW_BLK, H), lambda i: (i, 0))
    return pl.pallas_call(
        _mlp3_head_body,
        grid=(N // ROW_BLK,),
        in_specs=[blk,
                  pl.BlockSpec((NC, ROW_BLK, H), lambda i: (0, i, 0)),
                  full((H, H)), full((1, H)), full((H, H)), full((1, H)),
                  blk, blk,
                  full((3 * H, H)), full((1, H)),
                  full((H, H)), full((1, H)),
                  full((1, H)), full((1, H)), full((1, H)), full((1, H)),
                  full((H, H)), full((1, H))],
        out_specs=blk,
        out_shape=jax.ShapeDtypeStruct((N, H), jnp.float32),
    )(h, agg, W1, row(b1), W2, row(b2), h1, h2, W_jk, row(b_jk), Wc1, row(bc1),
      row(gam), row(bet), row(mu), row(var), Wc2, row(bc2))


def kernel(x, edge_index, W1_0, b1_0, W2_0, b2_0, W1_1, b1_1, W2_1, b2_1,
           W1_2, b1_2, W2_2, b2_2, W_jk, b_jk, Wc1, bc1,
           bn_gamma, bn_beta, bn_mean, bn_var, Wc2, bc2):
    idx = edge_index.reshape(2, NW, 2, HALF, CHUNK)
    zeros = jnp.zeros((ROWS_PER_TILE, H), jnp.float32)

    h = x
    xs = []
    for (W1, b1, W2, b2) in ((W1_0, b1_0, W2_0, b2_0),
                             (W1_1, b1_1, W2_1, b2_1)):
        agg = _sc_segment_sum(h, idx, zeros)
        h = _tc_mlp(h, agg, W1, b1, W2, b2)
        xs.append(h)

    agg = _sc_segment_sum(h, idx, zeros)
    return _tc_mlp3_head(h, agg, W1_2, b1_2, W2_2, b2_2, xs[0], xs[1],
                         W_jk, b_jk, Wc1, bc1,
                         bn_gamma, bn_beta, bn_mean, bn_var, Wc2, bc2)


# speedup vs baseline: 1.3565x; 1.0232x over previous
import functools

import jax
import jax.numpy as jnp
from jax import lax
from jax.experimental import pallas as pl
from jax.experimental.pallas import tpu as pltpu
from jax.experimental.pallas import tpu_sc as plsc

N = 10000
E = 320000
H = 128
BN_EPS = 1e-5

NC = 2
NS = 16
NW = NC * NS
EDGES_PER_W = E // NW
CHUNK = 125
CHUNKS = EDGES_PER_W // CHUNK
HALF = CHUNKS // 2
N_PAD = 10240
ROWS_PER_TILE = N_PAD // NS

_mesh = plsc.VectorSubcoreMesh(core_axis_name="c", subcore_axis_name="s")


@functools.partial(
    pl.kernel,
    out_type=jax.ShapeDtypeStruct((NC, N_PAD, H), jnp.float32),
    mesh=_mesh,
    scratch_types=[
        pltpu.VMEM((HALF, CHUNK), jnp.int32),
        pltpu.VMEM((HALF, CHUNK), jnp.int32),
        pltpu.VMEM((CHUNK, H), jnp.float32),
        pltpu.VMEM((CHUNK, H), jnp.float32),
        pltpu.VMEM_SHARED((N_PAD, H), jnp.float32),
        pltpu.SemaphoreType.DMA,
        pltpu.SemaphoreType.DMA,
        pltpu.SemaphoreType.DMA,
    ],
)
def _sc_segment_sum(h_hbm, idx_hbm, zeros_hbm, out_hbm,
                    src_v, dst_v, rows0_v, rows1_v, acc_sh,
                    sem_g0, sem_g1, sem_z):
    cid = lax.axis_index("c")
    sid = lax.axis_index("s")
    wid = sid * NC + cid

    zero_cp = pltpu.async_copy(
        zeros_hbm, acc_sh.at[pl.ds(sid * ROWS_PER_TILE, ROWS_PER_TILE)], sem_z)

    for p in range(2):
        pltpu.sync_copy(idx_hbm.at[0, wid, p], src_v)
        pltpu.sync_copy(idx_hbm.at[1, wid, p], dst_v)
        pltpu.async_copy(h_hbm.at[src_v.at[0]], rows0_v, sem_g0)
        pltpu.async_copy(h_hbm.at[src_v.at[1]], rows1_v, sem_g1)
        if p == 0:
            zero_cp.wait()
            plsc.subcore_barrier()

        def body(i, carry):
            j = i * 2
            pltpu.make_async_copy(h_hbm.at[src_v.at[0]], rows0_v, sem_g0).wait()
            pltpu.sync_copy(rows0_v, acc_sh.at[dst_v.at[j]], add=True)

            @pl.when(j + 2 < HALF)
            def _():
                pltpu.async_copy(h_hbm.at[src_v.at[j + 2]], rows0_v, sem_g0)

            pltpu.make_async_copy(h_hbm.at[src_v.at[0]], rows1_v, sem_g1).wait()
            pltpu.sync_copy(rows1_v, acc_sh.at[dst_v.at[j + 1]], add=True)

            @pl.when(j + 3 < HALF)
            def _():
                pltpu.async_copy(h_hbm.at[src_v.at[j + 3]], rows1_v, sem_g1)

            return carry

        lax.fori_loop(0, HALF // 2, body, 0)

    plsc.subcore_barrier()
    pltpu.sync_copy(acc_sh.at[pl.ds(sid * ROWS_PER_TILE, ROWS_PER_TILE)],
                    out_hbm.at[cid, pl.ds(sid * ROWS_PER_TILE, ROWS_PER_TILE)])


ROW_BLK = 2000


def _mlp_body(h_ref, agg_ref, w1_ref, b1_ref, w2_ref, b2_ref, o_ref):
    z = h_ref[...] + agg_ref[0] + agg_ref[1]
    t = jnp.dot(z, w1_ref[...], preferred_element_type=jnp.float32) + b1_ref[...]
    t = jnp.maximum(t, 0.0)
    t = jnp.dot(t, w2_ref[...], preferred_element_type=jnp.float32) + b2_ref[...]
    o_ref[...] = jnp.maximum(t, 0.0)


def _tc_mlp(h, agg, W1, b1, W2, b2):
    return pl.pallas_call(
        _mlp_body,
        grid=(N // ROW_BLK,),
        in_specs=[
            pl.BlockSpec((ROW_BLK, H), lambda i: (i, 0)),
            pl.BlockSpec((NC, ROW_BLK, H), lambda i: (0, i, 0)),
            pl.BlockSpec((H, H), lambda i: (0, 0)),
            pl.BlockSpec((1, H), lambda i: (0, 0)),
            pl.BlockSpec((H, H), lambda i: (0, 0)),
            pl.BlockSpec((1, H), lambda i: (0, 0)),
        ],
        out_specs=pl.BlockSpec((ROW_BLK, H), lambda i: (i, 0)),
        out_shape=jax.ShapeDtypeStruct((N, H), jnp.float32),
    )(h, agg, W1, b1.reshape(1, H), W2, b2.reshape(1, H))


def _mlp3_head_body(h_ref, agg_ref, w1_ref, b1_ref, w2_ref, b2_ref,
                    h1_ref, h2_ref, wjk_ref, bjk_ref, wc1_ref, bc1_ref,
                    gam_ref, bet_ref, mu_ref, var_ref, wc2_ref, bc2_ref, o_ref):
    z = h_ref[...] + agg_ref[0] + agg_ref[1]
    t = jnp.dot(z, w1_ref[...], preferred_element_type=jnp.float32) + b1_ref[...]
    t = jnp.maximum(t, 0.0)
    t = jnp.dot(t, w2_ref[...], preferred_element_type=jnp.float32) + b2_ref[...]
    h3 = jnp.maximum(t, 0.0)
    t = jnp.dot(h1_ref[...], wjk_ref[0:H, :], preferred_element_type=jnp.float32)
    t += jnp.dot(h2_ref[...], wjk_ref[H:2 * H, :], preferred_element_type=jnp.float32)
    t += jnp.dot(h3, wjk_ref[2 * H:3 * H, :], preferred_element_type=jnp.float32)
    t += bjk_ref[...]
    u = jnp.dot(t, wc1_ref[...], preferred_element_type=jnp.float32) + bc1_ref[...]
    scale = gam_ref[...] * lax.rsqrt(var_ref[...] + BN_EPS)
    u = (u - mu_ref[...]) * scale + bet_ref[...]
    u = jnp.maximum(u, 0.0)
    o_ref[...] = jnp.dot(u, wc2_ref[...], preferred_element_type=jnp.float32) + bc2_ref[...]


def _tc_mlp3_head(h, agg, W1, b1, W2, b2, h1, h2, W_jk, b_jk, Wc1, bc1,
                  gam, bet, mu, var, Wc2, bc2):
    row = lambda v: v.reshape(1, H)
    full = lambda shape: pl.BlockSpec(shape, lambda i: (0,) * len(shape))
    blk = pl.BlockSpec((ROW_BLK, H), lambda i: (i, 0))
    return pl.pallas_call(
        _mlp3_head_body,
        grid=(N // ROW_BLK,),
        in_specs=[blk,
                  pl.BlockSpec((NC, ROW_BLK, H), lambda i: (0, i, 0)),
                  full((H, H)), full((1, H)), full((H, H)), full((1, H)),
                  blk, blk,
                  full((3 * H, H)), full((1, H)),
                  full((H, H)), full((1, H)),
                  full((1, H)), full((1, H)), full((1, H)), full((1, H)),
                  full((H, H)), full((1, H))],
        out_specs=blk,
        out_shape=jax.ShapeDtypeStruct((N, H), jnp.float32),
    )(h, agg, W1, row(b1), W2, row(b2), h1, h2, W_jk, row(b_jk), Wc1, row(bc1),
      row(gam), row(bet), row(mu), row(var), Wc2, row(bc2))


def kernel(x, edge_index, W1_0, b1_0, W2_0, b2_0, W1_1, b1_1, W2_1, b2_1,
           W1_2, b1_2, W2_2, b2_2, W_jk, b_jk, Wc1, bc1,
           bn_gamma, bn_beta, bn_mean, bn_var, Wc2, bc2):
    idx = edge_index.reshape(2, NW, 2, HALF, CHUNK)
    zeros = jnp.zeros((ROWS_PER_TILE, H), jnp.float32)

    h = x
    xs = []
    for (W1, b1, W2, b2) in ((W1_0, b1_0, W2_0, b2_0),
                             (W1_1, b1_1, W2_1, b2_1)):
        agg = _sc_segment_sum(h, idx, zeros)
        h = _tc_mlp(h, agg, W1, b1, W2, b2)
        xs.append(h)

    agg = _sc_segment_sum(h, idx, zeros)
    return _tc_mlp3_head(h, agg, W1_2, b1_2, W2_2, b2_2, xs[0], xs[1],
                         W_jk, b_jk, Wc1, bc1,
                         bn_gamma, bn_beta, bn_mean, bn_var, Wc2, bc2)


# speedup vs baseline: 1.3823x; 1.0190x over previous
import functools

import jax
import jax.numpy as jnp
from jax import lax
from jax.experimental import pallas as pl
from jax.experimental.pallas import tpu as pltpu
from jax.experimental.pallas import tpu_sc as plsc

N = 10000
E = 320000
H = 128
BN_EPS = 1e-5

NC = 2
NS = 16
NW = NC * NS
EDGES_PER_W = E // NW
CHUNK = 125
CHUNKS = EDGES_PER_W // CHUNK
HALF = CHUNKS // 2
N_PAD = 10240
ROWS_PER_TILE = N_PAD // NS

_mesh = plsc.VectorSubcoreMesh(core_axis_name="c", subcore_axis_name="s")


@functools.partial(
    pl.kernel,
    out_type=jax.ShapeDtypeStruct((NC, N_PAD, H), jnp.float32),
    mesh=_mesh,
    scratch_types=[
        pltpu.VMEM((HALF, CHUNK), jnp.int32),
        pltpu.VMEM((HALF, CHUNK), jnp.int32),
        pltpu.VMEM((CHUNK, H), jnp.float32),
        pltpu.VMEM((CHUNK, H), jnp.float32),
        pltpu.VMEM_SHARED((N_PAD, H), jnp.float32),
        pltpu.SemaphoreType.DMA,
        pltpu.SemaphoreType.DMA,
        pltpu.SemaphoreType.DMA,
    ],
)
def _sc_segment_sum(h_hbm, idx_hbm, zeros_hbm, out_hbm,
                    src_v, dst_v, rows0_v, rows1_v, acc_sh,
                    sem_g0, sem_g1, sem_z):
    cid = lax.axis_index("c")
    sid = lax.axis_index("s")
    wid = sid * NC + cid

    base = sid * ROWS_PER_TILE

    @pl.when(jnp.logical_and(cid == 0, sid < NS - 1))
    def _():
        pltpu.async_copy(h_hbm.at[pl.ds(base, ROWS_PER_TILE)],
                         acc_sh.at[pl.ds(base, ROWS_PER_TILE)], sem_z)

    @pl.when(jnp.logical_and(cid == 0, sid == NS - 1))
    def _():
        pltpu.async_copy(h_hbm.at[pl.ds(base, N - (NS - 1) * ROWS_PER_TILE)],
                         acc_sh.at[pl.ds(base, N - (NS - 1) * ROWS_PER_TILE)],
                         sem_z)

    @pl.when(cid == 1)
    def _():
        pltpu.async_copy(zeros_hbm,
                         acc_sh.at[pl.ds(base, ROWS_PER_TILE)], sem_z)

    for p in range(2):
        pltpu.sync_copy(idx_hbm.at[0, wid, p], src_v)
        pltpu.sync_copy(idx_hbm.at[1, wid, p], dst_v)
        pltpu.async_copy(h_hbm.at[src_v.at[0]], rows0_v, sem_g0)
        pltpu.async_copy(h_hbm.at[src_v.at[1]], rows1_v, sem_g1)
        if p == 0:
            @pl.when(jnp.logical_and(cid == 0, sid < NS - 1))
            def _():
                pltpu.make_async_copy(
                    h_hbm.at[pl.ds(base, ROWS_PER_TILE)],
                    acc_sh.at[pl.ds(base, ROWS_PER_TILE)], sem_z).wait()

            @pl.when(jnp.logical_and(cid == 0, sid == NS - 1))
            def _():
                pltpu.make_async_copy(
                    h_hbm.at[pl.ds(base, N - (NS - 1) * ROWS_PER_TILE)],
                    acc_sh.at[pl.ds(base, N - (NS - 1) * ROWS_PER_TILE)],
                    sem_z).wait()

            @pl.when(cid == 1)
            def _():
                pltpu.make_async_copy(
                    zeros_hbm,
                    acc_sh.at[pl.ds(base, ROWS_PER_TILE)], sem_z).wait()

            plsc.subcore_barrier()

        def body(i, carry):
            j = i * 2
            pltpu.make_async_copy(h_hbm.at[src_v.at[0]], rows0_v, sem_g0).wait()
            pltpu.sync_copy(rows0_v, acc_sh.at[dst_v.at[j]], add=True)

            @pl.when(j + 2 < HALF)
            def _():
                pltpu.async_copy(h_hbm.at[src_v.at[j + 2]], rows0_v, sem_g0)

            pltpu.make_async_copy(h_hbm.at[src_v.at[0]], rows1_v, sem_g1).wait()
            pltpu.sync_copy(rows1_v, acc_sh.at[dst_v.at[j + 1]], add=True)

            @pl.when(j + 3 < HALF)
            def _():
                pltpu.async_copy(h_hbm.at[src_v.at[j + 3]], rows1_v, sem_g1)

            return carry

        lax.fori_loop(0, HALF // 2, body, 0)

    plsc.subcore_barrier()
    pltpu.sync_copy(acc_sh.at[pl.ds(sid * ROWS_PER_TILE, ROWS_PER_TILE)],
                    out_hbm.at[cid, pl.ds(sid * ROWS_PER_TILE, ROWS_PER_TILE)])


ROW_BLK = 2000


def _mlp_body(agg_ref, w1_ref, b1_ref, w2_ref, b2_ref, o_ref):
    z = agg_ref[0] + agg_ref[1]
    t = jnp.dot(z, w1_ref[...], preferred_element_type=jnp.float32) + b1_ref[...]
    t = jnp.maximum(t, 0.0)
    t = jnp.dot(t, w2_ref[...], preferred_element_type=jnp.float32) + b2_ref[...]
    o_ref[...] = jnp.maximum(t, 0.0)


def _tc_mlp(agg, W1, b1, W2, b2):
    return pl.pallas_call(
        _mlp_body,
        grid=(N // ROW_BLK,),
        in_specs=[
            pl.BlockSpec((NC, ROW_BLK, H), lambda i: (0, i, 0)),
            pl.BlockSpec((H, H), lambda i: (0, 0)),
            pl.BlockSpec((1, H), lambda i: (0, 0)),
            pl.BlockSpec((H, H), lambda i: (0, 0)),
            pl.BlockSpec((1, H), lambda i: (0, 0)),
        ],
        out_specs=pl.BlockSpec((ROW_BLK, H), lambda i: (i, 0)),
        out_shape=jax.ShapeDtypeStruct((N, H), jnp.float32),
    )(agg, W1, b1.reshape(1, H), W2, b2.reshape(1, H))


def _mlp3_head_body(agg_ref, w1_ref, b1_ref, w2_ref, b2_ref,
                    h1_ref, h2_ref, wjk_ref, bjk_ref, wc1_ref, bc1_ref,
                    gam_ref, bet_ref, mu_ref, var_ref, wc2_ref, bc2_ref, o_ref):
    z = agg_ref[0] + agg_ref[1]
    t = jnp.dot(z, w1_ref[...], preferred_element_type=jnp.float32) + b1_ref[...]
    t = jnp.maximum(t, 0.0)
    t = jnp.dot(t, w2_ref[...], preferred_element_type=jnp.float32) + b2_ref[...]
    h3 = jnp.maximum(t, 0.0)
    t = jnp.dot(h1_ref[...], wjk_ref[0:H, :], preferred_element_type=jnp.float32)
    t += jnp.dot(h2_ref[...], wjk_ref[H:2 * H, :], preferred_element_type=jnp.float32)
    t += jnp.dot(h3, wjk_ref[2 * H:3 * H, :], preferred_element_type=jnp.float32)
    t += bjk_ref[...]
    u = jnp.dot(t, wc1_ref[...], preferred_element_type=jnp.float32) + bc1_ref[...]
    scale = gam_ref[...] * lax.rsqrt(var_ref[...] + BN_EPS)
    u = (u - mu_ref[...]) * scale + bet_ref[...]
    u = jnp.maximum(u, 0.0)
    o_ref[...] = jnp.dot(u, wc2_ref[...], preferred_element_type=jnp.float32) + bc2_ref[...]


def _tc_mlp3_head(agg, W1, b1, W2, b2, h1, h2, W_jk, b_jk, Wc1, bc1,
                  gam, bet, mu, var, Wc2, bc2):
    row = lambda v: v.reshape(1, H)
    full = lambda shape: pl.BlockSpec(shape, lambda i: (0,) * len(shape))
    blk = pl.BlockSpec((ROW_BLK, H), lambda i: (i, 0))
    return pl.pallas_call(
        _mlp3_head_body,
        grid=(N // ROW_BLK,),
        in_specs=[pl.BlockSpec((NC, ROW_BLK, H), lambda i: (0, i, 0)),
                  full((H, H)), full((1, H)), full((H, H)), full((1, H)),
                  blk, blk,
                  full((3 * H, H)), full((1, H)),
                  full((H, H)), full((1, H)),
                  full((1, H)), full((1, H)), full((1, H)), full((1, H)),
                  full((H, H)), full((1, H))],
        out_specs=blk,
        out_shape=jax.ShapeDtypeStruct((N, H), jnp.float32),
    )(agg, W1, row(b1), W2, row(b2), h1, h2, W_jk, row(b_jk), Wc1, row(bc1),
      row(gam), row(bet), row(mu), row(var), Wc2, row(bc2))


def kernel(x, edge_index, W1_0, b1_0, W2_0, b2_0, W1_1, b1_1, W2_1, b2_1,
           W1_2, b1_2, W2_2, b2_2, W_jk, b_jk, Wc1, bc1,
           bn_gamma, bn_beta, bn_mean, bn_var, Wc2, bc2):
    idx = edge_index.reshape(2, NW, 2, HALF, CHUNK)
    zeros = jnp.zeros((ROWS_PER_TILE, H), jnp.float32)

    h = x
    xs = []
    for (W1, b1, W2, b2) in ((W1_0, b1_0, W2_0, b2_0),
                             (W1_1, b1_1, W2_1, b2_1)):
        agg = _sc_segment_sum(h, idx, zeros)
        h = _tc_mlp(agg, W1, b1, W2, b2)
        xs.append(h)

    agg = _sc_segment_sum(h, idx, zeros)
    return _tc_mlp3_head(agg, W1_2, b1_2, W2_2, b2_2, xs[0], xs[1],
                         W_jk, b_jk, Wc1, bc1,
                         bn_gamma, bn_beta, bn_mean, bn_var, Wc2, bc2)


# speedup vs baseline: 1.3861x; 1.0028x over previous
import functools

import jax
import jax.numpy as jnp
from jax import lax
from jax.experimental import pallas as pl
from jax.experimental.pallas import tpu as pltpu
from jax.experimental.pallas import tpu_sc as plsc

N = 10000
E = 320000
H = 128
BN_EPS = 1e-5

NC = 2
NS = 16
NW = NC * NS
EDGES_PER_W = E // NW
CHUNK = 125
CHUNKS = EDGES_PER_W // CHUNK
HALF = CHUNKS // 2
N_PAD = 10240
ROWS_PER_TILE = N_PAD // NS

_mesh = plsc.VectorSubcoreMesh(core_axis_name="c", subcore_axis_name="s")


@functools.partial(
    pl.kernel,
    out_type=jax.ShapeDtypeStruct((NC, N_PAD, H), jnp.float32),
    mesh=_mesh,
    scratch_types=[
        pltpu.VMEM((HALF, CHUNK), jnp.int32),
        pltpu.VMEM((HALF, CHUNK), jnp.int32),
        pltpu.VMEM((CHUNK, H), jnp.float32),
        pltpu.VMEM((CHUNK, H), jnp.float32),
        pltpu.VMEM_SHARED((N_PAD, H), jnp.float32),
        pltpu.SemaphoreType.DMA,
        pltpu.SemaphoreType.DMA,
        pltpu.SemaphoreType.DMA,
    ],
)
def _sc_segment_sum(h_hbm, idx_hbm, zeros_hbm, out_hbm,
                    src_v, dst_v, rows0_v, rows1_v, acc_sh,
                    sem_g0, sem_g1, sem_z):
    cid = lax.axis_index("c")
    sid = lax.axis_index("s")
    wid = sid * NC + cid

    base = sid * ROWS_PER_TILE

    @pl.when(jnp.logical_and(cid == 0, sid < NS - 1))
    def _():
        pltpu.async_copy(h_hbm.at[pl.ds(base, ROWS_PER_TILE)],
                         acc_sh.at[pl.ds(base, ROWS_PER_TILE)], sem_z)

    @pl.when(jnp.logical_and(cid == 0, sid == NS - 1))
    def _():
        pltpu.async_copy(h_hbm.at[pl.ds(base, N - (NS - 1) * ROWS_PER_TILE)],
                         acc_sh.at[pl.ds(base, N - (NS - 1) * ROWS_PER_TILE)],
                         sem_z)

    @pl.when(cid == 1)
    def _():
        pltpu.async_copy(zeros_hbm,
                         acc_sh.at[pl.ds(base, ROWS_PER_TILE)], sem_z)

    for p in range(2):
        pltpu.sync_copy(idx_hbm.at[0, wid, p], src_v)
        pltpu.sync_copy(idx_hbm.at[1, wid, p], dst_v)
        pltpu.async_copy(h_hbm.at[src_v.at[0]], rows0_v, sem_g0)
        pltpu.async_copy(h_hbm.at[src_v.at[1]], rows1_v, sem_g1)
        if p == 0:
            @pl.when(jnp.logical_and(cid == 0, sid < NS - 1))
            def _():
                pltpu.make_async_copy(
                    h_hbm.at[pl.ds(base, ROWS_PER_TILE)],
                    acc_sh.at[pl.ds(base, ROWS_PER_TILE)], sem_z).wait()

            @pl.when(jnp.logical_and(cid == 0, sid == NS - 1))
            def _():
                pltpu.make_async_copy(
                    h_hbm.at[pl.ds(base, N - (NS - 1) * ROWS_PER_TILE)],
                    acc_sh.at[pl.ds(base, N - (NS - 1) * ROWS_PER_TILE)],
                    sem_z).wait()

            @pl.when(cid == 1)
            def _():
                pltpu.make_async_copy(
                    zeros_hbm,
                    acc_sh.at[pl.ds(base, ROWS_PER_TILE)], sem_z).wait()

            plsc.subcore_barrier()

        def body(i, carry):
            j = i * 2
            pltpu.make_async_copy(h_hbm.at[src_v.at[0]], rows0_v, sem_g0).wait()
            pltpu.sync_copy(rows0_v, acc_sh.at[dst_v.at[j]], add=True)

            @pl.when(j + 2 < HALF)
            def _():
                pltpu.async_copy(h_hbm.at[src_v.at[j + 2]], rows0_v, sem_g0)

            pltpu.make_async_copy(h_hbm.at[src_v.at[0]], rows1_v, sem_g1).wait()
            pltpu.sync_copy(rows1_v, acc_sh.at[dst_v.at[j + 1]], add=True)

            @pl.when(j + 3 < HALF)
            def _():
                pltpu.async_copy(h_hbm.at[src_v.at[j + 3]], rows1_v, sem_g1)

            return carry

        lax.fori_loop(0, HALF // 2, body, 0)

    plsc.subcore_barrier()
    pltpu.sync_copy(acc_sh.at[pl.ds(sid * ROWS_PER_TILE, ROWS_PER_TILE)],
                    out_hbm.at[cid, pl.ds(sid * ROWS_PER_TILE, ROWS_PER_TILE)])


ROW_BLK = 5000


def _mlp_body(agg_ref, w1_ref, b1_ref, w2_ref, b2_ref, o_ref):
    z = agg_ref[0] + agg_ref[1]
    t = jnp.dot(z, w1_ref[...], preferred_element_type=jnp.float32) + b1_ref[...]
    t = jnp.maximum(t, 0.0)
    t = jnp.dot(t, w2_ref[...], preferred_element_type=jnp.float32) + b2_ref[...]
    o_ref[...] = jnp.maximum(t, 0.0)


def _tc_mlp(agg, W1, b1, W2, b2):
    return pl.pallas_call(
        _mlp_body,
        grid=(N // ROW_BLK,),
        in_specs=[
            pl.BlockSpec((NC, ROW_BLK, H), lambda i: (0, i, 0)),
            pl.BlockSpec((H, H), lambda i: (0, 0)),
            pl.BlockSpec((1, H), lambda i: (0, 0)),
            pl.BlockSpec((H, H), lambda i: (0, 0)),
            pl.BlockSpec((1, H), lambda i: (0, 0)),
        ],
        out_specs=pl.BlockSpec((ROW_BLK, H), lambda i: (i, 0)),
        out_shape=jax.ShapeDtypeStruct((N, H), jnp.float32),
    )(agg, W1, b1.reshape(1, H), W2, b2.reshape(1, H))


def _mlp3_head_body(agg_ref, w1_ref, b1_ref, w2_ref, b2_ref,
                    h1_ref, h2_ref, wjk_ref, bjk_ref, wc1_ref, bc1_ref,
                    gam_ref, bet_ref, mu_ref, var_ref, wc2_ref, bc2_ref, o_ref):
    z = agg_ref[0] + agg_ref[1]
    t = jnp.dot(z, w1_ref[...], preferred_element_type=jnp.float32) + b1_ref[...]
    t = jnp.maximum(t, 0.0)
    t = jnp.dot(t, w2_ref[...], preferred_element_type=jnp.float32) + b2_ref[...]
    h3 = jnp.maximum(t, 0.0)
    t = jnp.dot(h1_ref[...], wjk_ref[0:H, :], preferred_element_type=jnp.float32)
    t += jnp.dot(h2_ref[...], wjk_ref[H:2 * H, :], preferred_element_type=jnp.float32)
    t += jnp.dot(h3, wjk_ref[2 * H:3 * H, :], preferred_element_type=jnp.float32)
    t += bjk_ref[...]
    u = jnp.dot(t, wc1_ref[...], preferred_element_type=jnp.float32) + bc1_ref[...]
    scale = gam_ref[...] * lax.rsqrt(var_ref[...] + BN_EPS)
    u = (u - mu_ref[...]) * scale + bet_ref[...]
    u = jnp.maximum(u, 0.0)
    o_ref[...] = jnp.dot(u, wc2_ref[...], preferred_element_type=jnp.float32) + bc2_ref[...]


def _tc_mlp3_head(agg, W1, b1, W2, b2, h1, h2, W_jk, b_jk, Wc1, bc1,
                  gam, bet, mu, var, Wc2, bc2):
    row = lambda v: v.reshape(1, H)
    full = lambda shape: pl.BlockSpec(shape, lambda i: (0,) * len(shape))
    blk = pl.BlockSpec((ROW_BLK, H), lambda i: (i, 0))
    return pl.pallas_call(
        _mlp3_head_body,
        grid=(N // ROW_BLK,),
        in_specs=[pl.BlockSpec((NC, ROW_BLK, H), lambda i: (0, i, 0)),
                  full((H, H)), full((1, H)), full((H, H)), full((1, H)),
                  blk, blk,
                  full((3 * H, H)), full((1, H)),
                  full((H, H)), full((1, H)),
                  full((1, H)), full((1, H)), full((1, H)), full((1, H)),
                  full((H, H)), full((1, H))],
        out_specs=blk,
        out_shape=jax.ShapeDtypeStruct((N, H), jnp.float32),
    )(agg, W1, row(b1), W2, row(b2), h1, h2, W_jk, row(b_jk), Wc1, row(bc1),
      row(gam), row(bet), row(mu), row(var), Wc2, row(bc2))


def kernel(x, edge_index, W1_0, b1_0, W2_0, b2_0, W1_1, b1_1, W2_1, b2_1,
           W1_2, b1_2, W2_2, b2_2, W_jk, b_jk, Wc1, bc1,
           bn_gamma, bn_beta, bn_mean, bn_var, Wc2, bc2):
    idx = edge_index.reshape(2, NW, 2, HALF, CHUNK)
    zeros = jnp.zeros((ROWS_PER_TILE, H), jnp.float32)

    h = x
    xs = []
    for (W1, b1, W2, b2) in ((W1_0, b1_0, W2_0, b2_0),
                             (W1_1, b1_1, W2_1, b2_1)):
        agg = _sc_segment_sum(h, idx, zeros)
        h = _tc_mlp(agg, W1, b1, W2, b2)
        xs.append(h)

    agg = _sc_segment_sum(h, idx, zeros)
    return _tc_mlp3_head(agg, W1_2, b1_2, W2_2, b2_2, xs[0], xs[1],
                         W_jk, b_jk, Wc1, bc1,
                         bn_gamma, bn_beta, bn_mean, bn_var, Wc2, bc2)
